# baseline (speedup 1.0000x reference)
import functools

import jax
import jax.numpy as jnp
from jax import lax
from jax.experimental import pallas as pl
from jax.experimental.pallas import tpu as pltpu
from jax.experimental.pallas import tpu_sc as plsc

_NC = 2
_NS = 16
_NW = _NC * _NS


def _make_gather(n, da, dr):
    half = n // 2
    bpw = half // _NW
    ch = 512
    grp = bpw // ch
    k = ch // 128

    mesh = plsc.VectorSubcoreMesh(core_axis_name="c", subcore_axis_name="s")

    @functools.partial(
        pl.kernel,
        mesh=mesh,
        compiler_params=pltpu.CompilerParams(use_tc_tiling_on_sc=False),
        out_type=jax.ShapeDtypeStruct((half, 128), jnp.float32),
        scratch_types=[
            pltpu.VMEM((ch,), jnp.int32),
            pltpu.VMEM((ch,), jnp.int32),
            pltpu.VMEM((ch,), jnp.int32),
            pltpu.VMEM((ch,), jnp.int32),
            pltpu.VMEM((ch, da), jnp.float32),
            pltpu.VMEM((ch, da), jnp.float32),
            pltpu.VMEM((ch, dr), jnp.float32),
            pltpu.VMEM((ch, dr), jnp.float32),
            pltpu.SemaphoreType.DMA,
            pltpu.SemaphoreType.DMA,
        ],
    )
    def gather_k(aid_hbm, rid_hbm, at_hbm, rt_hbm, out,
                 aidx1, aidx2, ridx1, ridx2, a1, a2, r1, r2, sema, semr):
        wid = lax.axis_index("s") * _NC + lax.axis_index("c")

        def body(g, carry):
            base = pl.multiple_of(wid * bpw + g * ch, ch)
            base2 = base + half
            pltpu.sync_copy(aid_hbm.at[pl.ds(base, ch)], aidx1)
            pltpu.sync_copy(aid_hbm.at[pl.ds(base2, ch)], aidx2)
            pltpu.sync_copy(rid_hbm.at[pl.ds(base, ch)], ridx1)
            pltpu.sync_copy(rid_hbm.at[pl.ds(base2, ch)], ridx2)
            handles = []
            for j in range(k):
                sl = pl.ds(j * 128, 128)
                handles.append(pltpu.async_copy(
                    at_hbm.at[aidx1.at[sl]], a1.at[sl], sema))
                handles.append(pltpu.async_copy(
                    at_hbm.at[aidx2.at[sl]], a2.at[sl], sema))
                handles.append(pltpu.async_copy(
                    rt_hbm.at[ridx1.at[sl]], r1.at[sl], semr))
                handles.append(pltpu.async_copy(
                    rt_hbm.at[ridx2.at[sl]], r2.at[sl], semr))
            for h in handles:
                h.wait()
            rows = pl.ds(base, ch)
            pltpu.sync_copy(a1, out.at[rows, pl.ds(0, da)])
            pltpu.sync_copy(r1, out.at[rows, pl.ds(da, dr)])
            pltpu.sync_copy(a2, out.at[rows, pl.ds(64, da)])
            pltpu.sync_copy(r2, out.at[rows, pl.ds(64 + da, dr)])
            return carry

        lax.fori_loop(0, grp, body, 0)

    return gather_k


def _dense_body(x_ref, f1_ref, f2_ref, wc_ref, wf_ref, sh_ref, hh_ref,
                csum_ref, beff_ref, g2_ref, b2_ref, o_ref, *, total_in,
                da, dr, dm):
    x = x_ref[...]
    rblk = x.shape[0]
    nd = da + dr
    inv_n = 1.0 / float(total_in)
    inv_dm = 1.0 / dm
    lane = lax.broadcasted_iota(jnp.int32, (1, 128), 1)
    mfeat = jnp.logical_or(lane < nd,
                           jnp.logical_and(lane >= 64, lane < 64 + nd))
    xz = jnp.where(mfeat, x, 0.0)

    zpad = jnp.zeros((rblk, 8 - (total_in - nd)), jnp.float32)
    nf1 = jnp.concatenate(
        [jnp.log1p(jnp.maximum(f1_ref[...], 0.0)), zpad], axis=1)
    nf2 = jnp.concatenate(
        [jnp.log1p(jnp.maximum(f2_ref[...], 0.0)), zpad], axis=1)

    sh = sh_ref[...]
    xs = jnp.dot(xz, sh, preferred_element_type=jnp.float32)
    x2s = jnp.dot(xz * xz, sh, preferred_element_type=jnp.float32)
    nf1s = nf1[:, 0:1] + nf1[:, 1:2] + nf1[:, 2:3]
    nf2s = nf2[:, 0:1] + nf2[:, 1:2] + nf2[:, 2:3]
    nf1q = nf1 * nf1
    nf2q = nf2 * nf2
    nf1s2 = nf1q[:, 0:1] + nf1q[:, 1:2] + nf1q[:, 2:3]
    nf2s2 = nf2q[:, 0:1] + nf2q[:, 1:2] + nf2q[:, 2:3]
    mu1 = (xs[:, 0:1] + nf1s) * inv_n
    mu2 = (xs[:, 1:2] + nf2s) * inv_n
    var1 = (x2s[:, 0:1] + nf1s2) * inv_n - mu1 * mu1
    var2 = (x2s[:, 1:2] + nf2s2) * inv_n - mu2 * mu2
    inv1 = lax.rsqrt(jnp.maximum(var1, 0.0) + 1e-5)
    inv2 = lax.rsqrt(jnp.maximum(var2, 0.0) + 1e-5)

    y = jnp.dot(xz, wc_ref[...], preferred_element_type=jnp.float32)
    wf = wf_ref[...]
    csum = csum_ref[...]
    beff = beff_ref[...]
    y1 = y[:, 0:dm] + jnp.dot(nf1, wf, preferred_element_type=jnp.float32)
    y2 = y[:, dm:2 * dm] + jnp.dot(nf2, wf, preferred_element_type=jnp.float32)
    z1 = y1 * inv1 - (mu1 * inv1) * csum + beff
    z2 = y2 * inv2 - (mu2 * inv2) * csum + beff

    g2 = g2_ref[...]
    b2 = b2_ref[...]
    hh = hh_ref[...]
    for h, z in enumerate((z1, z2)):
        yg = 0.5 * z * (1.0 + lax.erf(z * 0.7071067811865476))
        s = jnp.dot(yg, hh, preferred_element_type=jnp.float32)
        s2 = jnp.dot(yg * yg, hh, preferred_element_type=jnp.float32)
        mu = s[:, 0:1] * inv_dm
        var = s2[:, 0:1] * inv_dm - mu * mu
        r = lax.rsqrt(jnp.maximum(var, 0.0) + 1e-5)
        o_ref[h] = (yg - mu) * (r * g2) + b2


def _dense(packed, nf, wc, wf, sh, hh, csum, beff, g2, b2, total_in, da, dr):
    half = packed.shape[0]
    nfd = nf.shape[1]
    dm = beff.shape[1]
    rblk = 1024
    nblk = half // rblk
    out = pl.pallas_call(
        functools.partial(_dense_body, total_in=total_in, da=da, dr=dr,
                          dm=dm),
        grid=(nblk,),
        in_specs=[
            pl.BlockSpec((rblk, 128), lambda i: (i, 0)),
            pl.BlockSpec((rblk, nfd), lambda i: (i, 0)),
            pl.BlockSpec((rblk, nfd), lambda i, _n=nblk: (i + _n, 0)),
            pl.BlockSpec((128, 2 * dm), lambda i: (0, 0)),
            pl.BlockSpec((8, dm), lambda i: (0, 0)),
            pl.BlockSpec((128, 8), lambda i: (0, 0)),
            pl.BlockSpec((dm, 8), lambda i: (0, 0)),
            pl.BlockSpec((1, dm), lambda i: (0, 0)),
            pl.BlockSpec((1, dm), lambda i: (0, 0)),
            pl.BlockSpec((1, dm), lambda i: (0, 0)),
            pl.BlockSpec((1, dm), lambda i: (0, 0)),
        ],
        out_specs=pl.BlockSpec((2, rblk, dm), lambda i: (0, i, 0)),
        out_shape=jax.ShapeDtypeStruct((2, half, dm), jnp.float32),
    )(packed, nf, nf, wc, wf, sh, hh, csum, beff, g2, b2)
    return out.reshape(2 * half, dm)


def kernel(act_ids, res_ids, num_feats, act_table, res_table,
           ln1_g, ln1_b, W, b, ln2_g, ln2_b):
    n = act_ids.shape[0]
    da = act_table.shape[1]
    dr = res_table.shape[1]
    nfd = num_feats.shape[1]
    dm = W.shape[1]
    nd = da + dr

    aid = act_ids.astype(jnp.int32)
    rid = res_ids.astype(jnp.int32)
    packed = _make_gather(n, da, dr)(aid, rid, act_table, res_table)

    wg = W * ln1_g[:, None]
    wc = jnp.zeros((128, 2 * dm), jnp.float32)
    wc = wc.at[0:nd, 0:dm].set(wg[0:nd])
    wc = wc.at[64:64 + nd, dm:2 * dm].set(wg[0:nd])
    wf = jnp.zeros((8, dm), jnp.float32).at[0:nfd].set(wg[nd:])
    sh = jnp.zeros((128, 8), jnp.float32)
    sh = sh.at[0:nd, 0].set(1.0)
    sh = sh.at[64:64 + nd, 1].set(1.0)
    hh = jnp.zeros((dm, 8), jnp.float32).at[:, 0].set(1.0)
    csum = jnp.sum(wg, axis=0).reshape(1, dm)
    beff = (ln1_b @ W + b).reshape(1, dm)
    g2 = ln2_g.reshape(1, dm)
    b2 = ln2_b.reshape(1, dm)
    return _dense(packed, num_feats, wc, wf, sh, hh, csum, beff, g2, b2,
                  da + dr + nfd, da, dr)
